# Initial kernel scaffold; baseline (speedup 1.0000x reference)
import functools

import jax
import jax.numpy as jnp
from jax import lax
from jax.experimental import pallas as pl
from jax.experimental.pallas import tpu as pltpu
from jax.experimental.pallas import tpu_sc as plsc


def _make_gather(n_rows: int, d: int):
    info = plsc.get_sparse_core_info()
    nc, ns = info.num_cores, info.num_subcores
    nw = nc * ns
    assert n_rows % nw == 0
    rows_per_w = n_rows // nw
    ch = 128
    assert rows_per_w % ch == 0
    n_chunks = rows_per_w // ch
    mesh = plsc.VectorSubcoreMesh(core_axis_name="c", subcore_axis_name="s")

    @functools.partial(
        pl.kernel,
        mesh=mesh,
        out_type=jax.ShapeDtypeStruct((n_rows, d), jnp.float32),
        scratch_types=[
            pltpu.VMEM((ch,), jnp.int32),
            pltpu.VMEM((ch, d), jnp.float32),
            pltpu.SemaphoreType.DMA,
        ],
    )
    def gather(ids_hbm, table_hbm, out_hbm, idx_v, rows_v, sem):
        wid = lax.axis_index("s") * nc + lax.axis_index("c")
        wbase = wid * rows_per_w

        def body(c, carry):
            base = wbase + c * ch
            pltpu.sync_copy(ids_hbm.at[pl.ds(base, ch)], idx_v)
            pltpu.async_copy(table_hbm.at[idx_v], rows_v, sem).wait()
            pltpu.sync_copy(rows_v, out_hbm.at[pl.ds(base, ch)])
            return carry

        lax.fori_loop(0, n_chunks, body, 0)

    return gather


def kernel(ids, table):
    b, s = ids.shape
    _, d = table.shape
    n = b * s
    flat = ids.reshape(n).astype(jnp.int32)
    out = _make_gather(n, d)(flat, table)
    return out.reshape(b, s, d)


# speedup vs baseline: 1.5727x; 1.5727x over previous
import functools

import jax
import jax.numpy as jnp
from jax import lax
from jax.experimental import pallas as pl
from jax.experimental.pallas import tpu as pltpu
from jax.experimental.pallas import tpu_sc as plsc


def _make_gather(n_rows: int, d: int):
    info = plsc.get_sparse_core_info()
    nc, ns = info.num_cores, info.num_subcores
    nw = nc * ns
    assert n_rows % nw == 0
    rows_per_w = n_rows // nw
    ch = 128
    assert rows_per_w % ch == 0
    n_chunks = rows_per_w // ch
    mesh = plsc.VectorSubcoreMesh(core_axis_name="c", subcore_axis_name="s")

    @functools.partial(
        pl.kernel,
        mesh=mesh,
        out_type=jax.ShapeDtypeStruct((n_rows, d), jnp.float32),
        scratch_types=[
            pltpu.VMEM((ch,), jnp.int32),
            pltpu.VMEM((ch, d), jnp.float32),
            pltpu.SemaphoreType.DMA,
        ],
        compiler_params=pltpu.CompilerParams(use_tc_tiling_on_sc=False),
    )
    def gather(ids_hbm, table_hbm, out_hbm, idx_v, rows_v, sem):
        wid = lax.axis_index("s") * nc + lax.axis_index("c")
        wbase = wid * rows_per_w

        def body(c, carry):
            base = wbase + c * ch
            pltpu.sync_copy(ids_hbm.at[pl.ds(base, ch)], idx_v)
            pltpu.async_copy(table_hbm.at[idx_v], rows_v, sem).wait()
            pltpu.sync_copy(rows_v, out_hbm.at[pl.ds(base, ch)])
            return carry

        lax.fori_loop(0, n_chunks, body, 0)

    return gather


def kernel(ids, table):
    b, s = ids.shape
    _, d = table.shape
    n = b * s
    flat = ids.reshape(n).astype(jnp.int32)
    out = _make_gather(n, d)(flat, table)
    return out.reshape(b, s, d)


# speedup vs baseline: 1.8705x; 1.1893x over previous
import functools

import jax
import jax.numpy as jnp
from jax import lax
from jax.experimental import pallas as pl
from jax.experimental.pallas import tpu as pltpu
from jax.experimental.pallas import tpu_sc as plsc

_CH = 128
_NBUF = 8


def _make_gather(n_rows: int, d: int):
    info = plsc.get_sparse_core_info()
    nc, ns = info.num_cores, info.num_subcores
    nw = nc * ns
    assert n_rows % (nw * _CH) == 0
    rows_per_w = n_rows // nw
    n_chunks_w = rows_per_w // _CH
    assert n_chunks_w % _NBUF == 0
    n_groups = n_chunks_w // _NBUF
    mesh = plsc.VectorSubcoreMesh(core_axis_name="c", subcore_axis_name="s")

    @functools.partial(
        pl.kernel,
        mesh=mesh,
        out_type=jax.ShapeDtypeStruct((n_rows, d), jnp.float32),
        scratch_types=[
            pltpu.VMEM((n_chunks_w, _CH), jnp.int32),
            pltpu.VMEM((_NBUF, _CH, d), jnp.float32),
            [pltpu.SemaphoreType.DMA] * _NBUF,
            [pltpu.SemaphoreType.DMA] * _NBUF,
        ],
        compiler_params=pltpu.CompilerParams(use_tc_tiling_on_sc=False),
    )
    def gather(ids_hbm, table_hbm, out_hbm, idx_v, rows_v, gsem, ssem):
        wid = lax.axis_index("s") * nc + lax.axis_index("c")
        wchunk0 = wid * n_chunks_w

        pltpu.sync_copy(ids_hbm.at[pl.ds(wchunk0, n_chunks_w)], idx_v)

        def g_start(c_local, b):
            pltpu.async_copy(table_hbm.at[idx_v.at[c_local]], rows_v.at[b],
                             gsem[b])

        def g_wait(b):
            pltpu.make_async_copy(table_hbm.at[idx_v.at[0]], rows_v.at[b],
                                  gsem[b]).wait()

        def s_start(c_local, b):
            base = (wchunk0 + c_local) * _CH
            pltpu.async_copy(rows_v.at[b], out_hbm.at[pl.ds(base, _CH)],
                             ssem[b])

        def s_wait(b):
            pltpu.make_async_copy(rows_v.at[b], out_hbm.at[pl.ds(0, _CH)],
                                  ssem[b]).wait()

        for b in range(_NBUF):
            g_start(b, b)

        def group(g, carry):
            for b in range(_NBUF):
                g_wait(b)
                s_start(g * _NBUF + b, b)
            for b in range(_NBUF):
                s_wait(b)
                g_start((g + 1) * _NBUF + b, b)
            return carry

        lax.fori_loop(0, n_groups - 1, group, 0)

        last = (n_groups - 1) * _NBUF
        for b in range(_NBUF):
            g_wait(b)
            s_start(last + b, b)
        for b in range(_NBUF):
            s_wait(b)

    return gather


def kernel(ids, table):
    b, s = ids.shape
    _, d = table.shape
    n = b * s
    flat = ids.reshape(n // _CH, _CH).astype(jnp.int32)
    out = _make_gather(n, d)(flat, table)
    return out.reshape(b, s, d)
